# baseline (speedup 1.0000x reference)
import functools

import jax
import jax.numpy as jnp
from jax import lax
from jax.experimental import pallas as pl
from jax.experimental.pallas import tpu as pltpu
from jax.experimental.pallas import tpu_sc as plsc

NC, NS = 2, 16
NW = NC * NS
CH = 80


def _mesh():
    return plsc.VectorSubcoreMesh(
        core_axis_name="c", subcore_axis_name="s", num_cores=NC, num_subcores=NS)


def _t_body(ea_ref, wb_ref, b_ref, t_ref, cs_ref, cs2_ref, acc_ref, acc2_ref):
    i = pl.program_id(0)
    t = jnp.dot(ea_ref[...], wb_ref[...], preferred_element_type=jnp.float32)
    t = t + b_ref[...]
    t_ref[...] = t

    @pl.when(i == 0)
    def _():
        acc_ref[...] = jnp.zeros_like(acc_ref)
        acc2_ref[...] = jnp.zeros_like(acc2_ref)

    acc_ref[...] += jnp.sum(t, axis=0, keepdims=True)
    acc2_ref[...] += jnp.sum(t * t, axis=0, keepdims=True)

    @pl.when(i == pl.num_programs(0) - 1)
    def _():
        cs_ref[...] = acc_ref[...]
        cs2_ref[...] = acc2_ref[...]


def _t_matmul(ea, wb, b1a, blk):
    E, H = ea.shape
    grid = E // blk
    return pl.pallas_call(
        _t_body,
        grid=(grid,),
        in_specs=[
            pl.BlockSpec((blk, H), lambda i: (i, 0)),
            pl.BlockSpec((H, H), lambda i: (0, 0)),
            pl.BlockSpec((1, H), lambda i: (0, 0)),
        ],
        out_specs=[
            pl.BlockSpec((blk, H), lambda i: (i, 0)),
            pl.BlockSpec((1, H), lambda i: (0, 0)),
            pl.BlockSpec((1, H), lambda i: (0, 0)),
        ],
        out_shape=[
            jax.ShapeDtypeStruct((E, H), jnp.float32),
            jax.ShapeDtypeStruct((1, H), jnp.float32),
            jax.ShapeDtypeStruct((1, H), jnp.float32),
        ],
        scratch_shapes=[
            pltpu.VMEM((1, H), jnp.float32),
            pltpu.VMEM((1, H), jnp.float32),
        ],
    )(ea, wb, b1a)


def _stats_body(einv, x_ref, w1a_ref, b1a_ref, g1_ref, be1_ref, sea_ref,
                cnt_ref, cs_ref, cs2_ref, p2_ref, scale_ref):
    D = x_ref.shape[1]
    N = x_ref.shape[0]
    wt = w1a_ref[0:D, :]
    wb = w1a_ref[D:, :]
    p = jnp.dot(x_ref[...], wt, preferred_element_type=jnp.float32)
    sea = sea_ref[...]
    cnt = cnt_ref[...]
    r = jnp.dot(sea, wb, preferred_element_type=jnp.float32) + cnt * b1a_ref[...]
    sh = jnp.sum(cnt * p, axis=0, keepdims=True) + cs_ref[...]
    sh2 = (jnp.sum(cnt * p * p, axis=0, keepdims=True)
           + 2.0 * jnp.sum(p * r, axis=0, keepdims=True) + cs2_ref[...])
    m = sh * einv
    var = sh2 * einv - m * m
    scale = g1_ref[...] * lax.rsqrt(var + 1e-5)
    shift = be1_ref[...] - m * scale
    p2_ref[...] = p * scale + shift
    scale_ref[...] = scale


def _stats(x, w1a, b1a, g1, be1, sea, cnt, cs, cs2, einv):
    N, D = x.shape
    H = w1a.shape[1]
    return pl.pallas_call(
        functools.partial(_stats_body, einv),
        out_shape=[
            jax.ShapeDtypeStruct((N, H), jnp.float32),
            jax.ShapeDtypeStruct((1, H), jnp.float32),
        ],
    )(x, w1a, b1a, g1, be1, sea, cnt, cs, cs2)


def _make_main(E, H):
    per_w = E // NW
    CB = 128
    chunks = per_w // CB
    rem = per_w - chunks * CB

    def body(t_hbm, row_hbm, p2_hbm, scale_hbm,
             y_out,
             t_v, pg_v, slab_v, scale_v, sem):
        c = lax.axis_index("c")
        s = lax.axis_index("s")
        w = c * NS + s

        pltpu.sync_copy(scale_hbm, scale_v)
        sv = [scale_v[pl.ds(16 * jj, 16)] for jj in range(H // 16)]

        def do_chunk(base, size):
            pltpu.sync_copy(row_hbm.at[pl.ds(base, size)],
                            slab_v.at[0, pl.ds(0, size)])
            gather = pltpu.async_copy(
                p2_hbm.at[slab_v.at[0, pl.ds(0, size)]],
                pg_v.at[pl.ds(0, size)], sem)
            pltpu.sync_copy(t_hbm.at[pl.ds(base, size)],
                            t_v.at[pl.ds(0, size)])
            gather.wait()

            def rowop(rr, _2):
                for jj in range(H // 16):
                    sl = pl.ds(16 * jj, 16)
                    pg_v[rr, sl] = jnp.maximum(
                        t_v[rr, sl] * sv[jj] + pg_v[rr, sl], 0.0)
                return _2
            lax.fori_loop(0, size, rowop, 0)

            pltpu.sync_copy(pg_v.at[pl.ds(0, size)],
                            y_out.at[pl.ds(base, size)])

        def step(j, _):
            do_chunk(w * per_w + j * CB, CB)
            return _
        lax.fori_loop(0, chunks, step, 0)
        if rem:
            do_chunk(w * per_w + chunks * CB, rem)

    return pl.kernel(
        body,
        out_type=jax.ShapeDtypeStruct((E, H), jnp.float32),
        mesh=_mesh(),
        scratch_types=[
            pltpu.VMEM((CB, H), jnp.float32),
            pltpu.VMEM((CB, H), jnp.float32),
            pltpu.VMEM((2, CB), jnp.int32),
            pltpu.VMEM((H,), jnp.float32),
            pltpu.SemaphoreType.DMA,
        ],
    )


def _final_body(s_ref, cnt_ref, x_ref, w1b_ref, b1b_ref, w2a_ref, b2a_ref,
                g2_ref, be2_ref, w2b_ref, b2b_ref, out_ref):
    N, D = x_ref.shape
    su = s_ref[...]
    cnt = cnt_ref[...]
    mean_in = su / jnp.maximum(cnt, 1.0)
    meanf = (jnp.dot(mean_in, w1b_ref[...], preferred_element_type=jnp.float32)
             + b1b_ref[...] * (cnt > 0.0).astype(jnp.float32))
    z = (jnp.dot(x_ref[...], w2a_ref[0:D, :], preferred_element_type=jnp.float32)
         + jnp.dot(meanf, w2a_ref[D:, :], preferred_element_type=jnp.float32)
         + b2a_ref[...])
    mz = jnp.mean(z, axis=0, keepdims=True)
    vz = jnp.mean(z * z, axis=0, keepdims=True) - mz * mz
    zb = (z - mz) * lax.rsqrt(vz + 1e-5) * g2_ref[...] + be2_ref[...]
    zb = jnp.maximum(zb, 0.0)
    out_ref[...] = (jnp.dot(zb, w2b_ref[...], preferred_element_type=jnp.float32)
                    + b2b_ref[...])


def _final(s, cnt, x, w1b, b1b, w2a, b2a, g2, be2, w2b, b2b):
    N, D = x.shape
    H = w1b.shape[1]
    return pl.pallas_call(
        _final_body,
        out_shape=jax.ShapeDtypeStruct((N, H), jnp.float32),
    )(s, cnt, x, w1b, b1b, w2a, b2a, g2, be2, w2b, b2b)


def kernel(x, edge_index, edge_attr, u, batch,
           W1a, b1a, g1, be1, W1b, b1b,
           W2a, b2a, g2, be2, W2b, b2b):
    N, D = x.shape
    E = edge_index.shape[1]
    H = edge_attr.shape[1]
    row = edge_index[0]
    col = edge_index[1]

    b1a2 = b1a.reshape(1, H)
    g12 = g1.reshape(1, H)
    be12 = be1.reshape(1, H)

    t, cs, cs2 = _t_matmul(edge_attr, W1a[D:], b1a2, blk=2000)

    ones_e = jnp.ones((E,), jnp.float32)
    sea = jax.ops.segment_sum(edge_attr, row, num_segments=N)
    cnt_row = jax.ops.segment_sum(ones_e, row, num_segments=N)[:, None]
    cnt_col = jax.ops.segment_sum(ones_e, col, num_segments=N)[:, None]

    einv = 1.0 / E
    p2, scale = _stats(x, W1a, b1a2, g12, be12, sea, cnt_row, cs, cs2, einv)

    y = _make_main(E, H)(t, row, p2, scale.reshape(H))
    s = jax.ops.segment_sum(y, col, num_segments=N)

    out = _final(s, cnt_col, x,
                 W1b, b1b.reshape(1, H), W2a, b2a.reshape(1, H),
                 g2.reshape(1, H), be2.reshape(1, H), W2b, b2b.reshape(1, H))
    return out


# speedup vs baseline: 1.3951x; 1.3951x over previous
import functools

import jax
import jax.numpy as jnp
from jax import lax
from jax.experimental import pallas as pl
from jax.experimental.pallas import tpu as pltpu
from jax.experimental.pallas import tpu_sc as plsc

NC, NS = 2, 16
NW = NC * NS
CH = 80


def _mesh():
    return plsc.VectorSubcoreMesh(
        core_axis_name="c", subcore_axis_name="s", num_cores=NC, num_subcores=NS)


def _t_body(ea_ref, wb_ref, b_ref, t_ref, cs_ref, cs2_ref, acc_ref, acc2_ref):
    i = pl.program_id(0)
    t = jnp.dot(ea_ref[...], wb_ref[...], preferred_element_type=jnp.float32)
    t = t + b_ref[...]
    t_ref[...] = t

    @pl.when(i == 0)
    def _():
        acc_ref[...] = jnp.zeros_like(acc_ref)
        acc2_ref[...] = jnp.zeros_like(acc2_ref)

    acc_ref[...] += jnp.sum(t, axis=0, keepdims=True)
    acc2_ref[...] += jnp.sum(t * t, axis=0, keepdims=True)

    @pl.when(i == pl.num_programs(0) - 1)
    def _():
        cs_ref[...] = acc_ref[...]
        cs2_ref[...] = acc2_ref[...]


def _t_matmul(ea, wb, b1a, blk):
    E, H = ea.shape
    grid = E // blk
    return pl.pallas_call(
        _t_body,
        grid=(grid,),
        in_specs=[
            pl.BlockSpec((blk, H), lambda i: (i, 0)),
            pl.BlockSpec((H, H), lambda i: (0, 0)),
            pl.BlockSpec((1, H), lambda i: (0, 0)),
        ],
        out_specs=[
            pl.BlockSpec((blk, H), lambda i: (i, 0)),
            pl.BlockSpec((1, H), lambda i: (0, 0)),
            pl.BlockSpec((1, H), lambda i: (0, 0)),
        ],
        out_shape=[
            jax.ShapeDtypeStruct((E, H), jnp.float32),
            jax.ShapeDtypeStruct((1, H), jnp.float32),
            jax.ShapeDtypeStruct((1, H), jnp.float32),
        ],
        scratch_shapes=[
            pltpu.VMEM((1, H), jnp.float32),
            pltpu.VMEM((1, H), jnp.float32),
        ],
    )(ea, wb, b1a)


def _pmat_body(x_ref, wt_ref, p_ref):
    p_ref[...] = jnp.dot(x_ref[...], wt_ref[...],
                         preferred_element_type=jnp.float32)


def _pmat(x, wt):
    N, D = x.shape
    H = wt.shape[1]
    return pl.pallas_call(
        _pmat_body,
        out_shape=jax.ShapeDtypeStruct((N, H), jnp.float32),
    )(x, wt)


def _make_hstats(E, H):
    per_w = E // NW
    CB = 128
    chunks = per_w // CB
    rem = per_w - chunks * CB
    NV = H // 16

    def body(t_hbm, row_hbm, p_hbm, hs_out, t_v, pg_v, slab_v, acc_v, sem):
        c = lax.axis_index("c")
        s = lax.axis_index("s")
        w = c * NS + s
        zero = jnp.zeros((16,), jnp.float32)

        def do_chunk(base, size, carry):
            pltpu.sync_copy(row_hbm.at[pl.ds(base, size)],
                            slab_v.at[0, pl.ds(0, size)])
            gather = pltpu.async_copy(
                p_hbm.at[slab_v.at[0, pl.ds(0, size)]],
                pg_v.at[pl.ds(0, size)], sem)
            pltpu.sync_copy(t_hbm.at[pl.ds(base, size)],
                            t_v.at[pl.ds(0, size)])
            gather.wait()

            def rowop(rr, cy):
                sums, sqs = cy
                ns, nq = [], []
                for jj in range(NV):
                    sl = pl.ds(16 * jj, 16)
                    h = t_v[rr, sl] + pg_v[rr, sl]
                    ns.append(sums[jj] + h)
                    nq.append(sqs[jj] + h * h)
                return ns, nq
            return lax.fori_loop(0, size, rowop, carry)

        def step(j, carry):
            return do_chunk(w * per_w + j * CB, CB, carry)
        carry = ([zero] * NV, [zero] * NV)
        carry = lax.fori_loop(0, chunks, step, carry)
        if rem:
            carry = do_chunk(w * per_w + chunks * CB, rem, carry)

        sums, sqs = carry
        for jj in range(NV):
            acc_v[pl.ds(16 * jj, 16)] = sums[jj]
            acc_v[pl.ds(H + 16 * jj, 16)] = sqs[jj]
        pltpu.sync_copy(acc_v, hs_out.at[w])

    return pl.kernel(
        body,
        out_type=jax.ShapeDtypeStruct((NW, 2 * H), jnp.float32),
        mesh=_mesh(),
        scratch_types=[
            pltpu.VMEM((CB, H), jnp.float32),
            pltpu.VMEM((CB, H), jnp.float32),
            pltpu.VMEM((2, CB), jnp.int32),
            pltpu.VMEM((2 * H,), jnp.float32),
            pltpu.SemaphoreType.DMA,
        ],
    )


def _stats_body(einv, H, p_ref, g1_ref, be1_ref, hs_ref, p2_ref, scale_ref):
    sh = jnp.sum(hs_ref[:, 0:H], axis=0, keepdims=True)
    sh2 = jnp.sum(hs_ref[:, H:], axis=0, keepdims=True)
    m = sh * einv
    var = sh2 * einv - m * m
    scale = g1_ref[...] * lax.rsqrt(var + 1e-5)
    shift = be1_ref[...] - m * scale
    p2_ref[...] = p_ref[...] * scale + shift
    scale_ref[...] = scale


def _stats(p, g1, be1, hs, einv):
    N, H = p.shape
    return pl.pallas_call(
        functools.partial(_stats_body, einv, H),
        out_shape=[
            jax.ShapeDtypeStruct((N, H), jnp.float32),
            jax.ShapeDtypeStruct((1, H), jnp.float32),
        ],
    )(p, g1, be1, hs)


def _make_main(E, H):
    per_w = E // NW
    CB = 128
    chunks = per_w // CB
    rem = per_w - chunks * CB

    def body(t_hbm, row_hbm, p2_hbm, scale_hbm,
             y_out,
             t_v, pg_v, slab_v, scale_v, sem):
        c = lax.axis_index("c")
        s = lax.axis_index("s")
        w = c * NS + s

        pltpu.sync_copy(scale_hbm, scale_v)
        sv = [scale_v[pl.ds(16 * jj, 16)] for jj in range(H // 16)]

        def do_chunk(base, size):
            pltpu.sync_copy(row_hbm.at[pl.ds(base, size)],
                            slab_v.at[0, pl.ds(0, size)])
            gather = pltpu.async_copy(
                p2_hbm.at[slab_v.at[0, pl.ds(0, size)]],
                pg_v.at[pl.ds(0, size)], sem)
            pltpu.sync_copy(t_hbm.at[pl.ds(base, size)],
                            t_v.at[pl.ds(0, size)])
            gather.wait()

            def rowop(rr, _2):
                for jj in range(H // 16):
                    sl = pl.ds(16 * jj, 16)
                    pg_v[rr, sl] = jnp.maximum(
                        t_v[rr, sl] * sv[jj] + pg_v[rr, sl], 0.0)
                return _2
            lax.fori_loop(0, size, rowop, 0)

            pltpu.sync_copy(pg_v.at[pl.ds(0, size)],
                            y_out.at[pl.ds(base, size)])

        def step(j, _):
            do_chunk(w * per_w + j * CB, CB)
            return _
        lax.fori_loop(0, chunks, step, 0)
        if rem:
            do_chunk(w * per_w + chunks * CB, rem)

    return pl.kernel(
        body,
        out_type=jax.ShapeDtypeStruct((E, H), jnp.float32),
        mesh=_mesh(),
        scratch_types=[
            pltpu.VMEM((CB, H), jnp.float32),
            pltpu.VMEM((CB, H), jnp.float32),
            pltpu.VMEM((2, CB), jnp.int32),
            pltpu.VMEM((H,), jnp.float32),
            pltpu.SemaphoreType.DMA,
        ],
    )


def _final_body(s_ref, cnt_ref, x_ref, w1b_ref, b1b_ref, w2a_ref, b2a_ref,
                g2_ref, be2_ref, w2b_ref, b2b_ref, out_ref):
    N, D = x_ref.shape
    su = s_ref[...]
    cnt = cnt_ref[...]
    mean_in = su / jnp.maximum(cnt, 1.0)
    meanf = (jnp.dot(mean_in, w1b_ref[...], preferred_element_type=jnp.float32)
             + b1b_ref[...] * (cnt > 0.0).astype(jnp.float32))
    z = (jnp.dot(x_ref[...], w2a_ref[0:D, :], preferred_element_type=jnp.float32)
         + jnp.dot(meanf, w2a_ref[D:, :], preferred_element_type=jnp.float32)
         + b2a_ref[...])
    mz = jnp.mean(z, axis=0, keepdims=True)
    vz = jnp.mean(z * z, axis=0, keepdims=True) - mz * mz
    zb = (z - mz) * lax.rsqrt(vz + 1e-5) * g2_ref[...] + be2_ref[...]
    zb = jnp.maximum(zb, 0.0)
    out_ref[...] = (jnp.dot(zb, w2b_ref[...], preferred_element_type=jnp.float32)
                    + b2b_ref[...])


def _final(s, cnt, x, w1b, b1b, w2a, b2a, g2, be2, w2b, b2b):
    N, D = x.shape
    H = w1b.shape[1]
    return pl.pallas_call(
        _final_body,
        out_shape=jax.ShapeDtypeStruct((N, H), jnp.float32),
    )(s, cnt, x, w1b, b1b, w2a, b2a, g2, be2, w2b, b2b)


def kernel(x, edge_index, edge_attr, u, batch,
           W1a, b1a, g1, be1, W1b, b1b,
           W2a, b2a, g2, be2, W2b, b2b):
    N, D = x.shape
    E = edge_index.shape[1]
    H = edge_attr.shape[1]
    row = edge_index[0]
    col = edge_index[1]

    b1a2 = b1a.reshape(1, H)
    g12 = g1.reshape(1, H)
    be12 = be1.reshape(1, H)

    t, cs, cs2 = _t_matmul(edge_attr, W1a[D:], b1a2, blk=2000)

    ones_e = jnp.ones((E,), jnp.float32)
    cnt_col = jax.ops.segment_sum(ones_e, col, num_segments=N)[:, None]

    p = _pmat(x, W1a[0:D])
    hs = _make_hstats(E, H)(t, row, p)

    einv = 1.0 / E
    p2, scale = _stats(p, g12, be12, hs, einv)

    y = _make_main(E, H)(t, row, p2, scale.reshape(H))
    s = jax.ops.segment_sum(y, col, num_segments=N)

    out = _final(s, cnt_col, x,
                 W1b, b1b.reshape(1, H), W2a, b2a.reshape(1, H),
                 g2.reshape(1, H), be2.reshape(1, H), W2b, b2b.reshape(1, H))
    return out


# speedup vs baseline: 1.4084x; 1.0096x over previous
import functools

import jax
import jax.numpy as jnp
from jax import lax
from jax.experimental import pallas as pl
from jax.experimental.pallas import tpu as pltpu
from jax.experimental.pallas import tpu_sc as plsc

NC, NS = 2, 16
NW = NC * NS


def _mesh():
    return plsc.VectorSubcoreMesh(
        core_axis_name="c", subcore_axis_name="s", num_cores=NC, num_subcores=NS)


def _t_body(ea_ref, wb_ref, b_ref, t_ref):
    t = jnp.dot(ea_ref[...], wb_ref[...], preferred_element_type=jnp.float32)
    t_ref[...] = t + b_ref[...]


def _t_matmul(ea, wb, b1a, blk):
    E, H = ea.shape
    grid = E // blk
    return pl.pallas_call(
        _t_body,
        grid=(grid,),
        in_specs=[
            pl.BlockSpec((blk, H), lambda i: (i, 0)),
            pl.BlockSpec((H, H), lambda i: (0, 0)),
            pl.BlockSpec((1, H), lambda i: (0, 0)),
        ],
        out_specs=pl.BlockSpec((blk, H), lambda i: (i, 0)),
        out_shape=jax.ShapeDtypeStruct((E, H), jnp.float32),
    )(ea, wb, b1a)


def _pmat_body(x_ref, wt_ref, p_ref):
    p_ref[...] = jnp.dot(x_ref[...], wt_ref[...],
                         preferred_element_type=jnp.float32)


def _pmat(x, wt):
    N, D = x.shape
    H = wt.shape[1]
    return pl.pallas_call(
        _pmat_body,
        out_shape=jax.ShapeDtypeStruct((N, H), jnp.float32),
    )(x, wt)


def _make_hstats(E, H):
    per_w = E // NW
    CB = 128
    chunks = per_w // CB
    rem = per_w - chunks * CB
    NV = H // 16

    def body(t_hbm, row_hbm, p_hbm, hs_out, t_v, pg_v, slab_v, acc_v, sem):
        c = lax.axis_index("c")
        s = lax.axis_index("s")
        w = c * NS + s
        zero = jnp.zeros((16,), jnp.float32)

        def do_chunk(base, size, carry):
            pltpu.sync_copy(row_hbm.at[pl.ds(base, size)],
                            slab_v.at[0, pl.ds(0, size)])
            gather = pltpu.async_copy(
                p_hbm.at[slab_v.at[0, pl.ds(0, size)]],
                pg_v.at[pl.ds(0, size)], sem)
            pltpu.sync_copy(t_hbm.at[pl.ds(base, size)],
                            t_v.at[pl.ds(0, size)])
            gather.wait()

            def rowop(rr, cy):
                sums, sqs = cy
                ns, nq = [], []
                for jj in range(NV):
                    sl = pl.ds(16 * jj, 16)
                    h = t_v[rr, sl] + pg_v[rr, sl]
                    ns.append(sums[jj] + h)
                    nq.append(sqs[jj] + h * h)
                return ns, nq
            return lax.fori_loop(0, size, rowop, carry)

        def step(j, carry):
            return do_chunk(w * per_w + j * CB, CB, carry)
        carry = ([zero] * NV, [zero] * NV)
        carry = lax.fori_loop(0, chunks, step, carry)
        if rem:
            carry = do_chunk(w * per_w + chunks * CB, rem, carry)

        sums, sqs = carry
        for jj in range(NV):
            acc_v[pl.ds(16 * jj, 16)] = sums[jj]
            acc_v[pl.ds(H + 16 * jj, 16)] = sqs[jj]
        pltpu.sync_copy(acc_v, hs_out.at[w])

    return pl.kernel(
        body,
        out_type=jax.ShapeDtypeStruct((NW, 2 * H), jnp.float32),
        mesh=_mesh(),
        scratch_types=[
            pltpu.VMEM((CB, H), jnp.float32),
            pltpu.VMEM((CB, H), jnp.float32),
            pltpu.VMEM((2, CB), jnp.int32),
            pltpu.VMEM((2 * H,), jnp.float32),
            pltpu.SemaphoreType.DMA,
        ],
    )


def _stats_body(einv, H, p_ref, g1_ref, be1_ref, hs_ref, p2_ref, scale_ref):
    sh = jnp.sum(hs_ref[:, 0:H], axis=0, keepdims=True)
    sh2 = jnp.sum(hs_ref[:, H:], axis=0, keepdims=True)
    m = sh * einv
    var = sh2 * einv - m * m
    scale = g1_ref[...] * lax.rsqrt(var + 1e-5)
    shift = be1_ref[...] - m * scale
    p2_ref[...] = p_ref[...] * scale + shift
    scale_ref[...] = scale


def _stats(p, g1, be1, hs, einv):
    N, H = p.shape
    return pl.pallas_call(
        functools.partial(_stats_body, einv, H),
        out_shape=[
            jax.ShapeDtypeStruct((N, H), jnp.float32),
            jax.ShapeDtypeStruct((1, H), jnp.float32),
        ],
    )(p, g1, be1, hs)


def _make_main(E, H):
    per_w = E // NW
    CB = 128
    chunks = per_w // CB
    rem = per_w - chunks * CB

    def body(t_hbm, row_hbm, p2_hbm, scale_hbm,
             y_out,
             t_v, pg_v, slab_v, scale_v, sem):
        c = lax.axis_index("c")
        s = lax.axis_index("s")
        w = c * NS + s

        pltpu.sync_copy(scale_hbm, scale_v)
        sv = [scale_v[pl.ds(16 * jj, 16)] for jj in range(H // 16)]

        def do_chunk(base, size):
            pltpu.sync_copy(row_hbm.at[pl.ds(base, size)],
                            slab_v.at[0, pl.ds(0, size)])
            gather = pltpu.async_copy(
                p2_hbm.at[slab_v.at[0, pl.ds(0, size)]],
                pg_v.at[pl.ds(0, size)], sem)
            pltpu.sync_copy(t_hbm.at[pl.ds(base, size)],
                            t_v.at[pl.ds(0, size)])
            gather.wait()

            def rowop(rr, _2):
                for jj in range(H // 16):
                    sl = pl.ds(16 * jj, 16)
                    pg_v[rr, sl] = jnp.maximum(
                        t_v[rr, sl] * sv[jj] + pg_v[rr, sl], 0.0)
                return _2
            lax.fori_loop(0, size, rowop, 0)

            pltpu.sync_copy(pg_v.at[pl.ds(0, size)],
                            y_out.at[pl.ds(base, size)])

        def step(j, _):
            do_chunk(w * per_w + j * CB, CB)
            return _
        lax.fori_loop(0, chunks, step, 0)
        if rem:
            do_chunk(w * per_w + chunks * CB, rem)

    return pl.kernel(
        body,
        out_type=jax.ShapeDtypeStruct((E, H), jnp.float32),
        mesh=_mesh(),
        scratch_types=[
            pltpu.VMEM((CB, H), jnp.float32),
            pltpu.VMEM((CB, H), jnp.float32),
            pltpu.VMEM((2, CB), jnp.int32),
            pltpu.VMEM((H,), jnp.float32),
            pltpu.SemaphoreType.DMA,
        ],
    )


def _final_body(s_ref, cnt_ref, x_ref, w1b_ref, b1b_ref, w2a_ref, b2a_ref,
                g2_ref, be2_ref, w2b_ref, b2b_ref, out_ref):
    N, D = x_ref.shape
    su = s_ref[...]
    cnt = cnt_ref[...]
    mean_in = su / jnp.maximum(cnt, 1.0)
    meanf = (jnp.dot(mean_in, w1b_ref[...], preferred_element_type=jnp.float32)
             + b1b_ref[...] * (cnt > 0.0).astype(jnp.float32))
    z = (jnp.dot(x_ref[...], w2a_ref[0:D, :], preferred_element_type=jnp.float32)
         + jnp.dot(meanf, w2a_ref[D:, :], preferred_element_type=jnp.float32)
         + b2a_ref[...])
    mz = jnp.mean(z, axis=0, keepdims=True)
    vz = jnp.mean(z * z, axis=0, keepdims=True) - mz * mz
    zb = (z - mz) * lax.rsqrt(vz + 1e-5) * g2_ref[...] + be2_ref[...]
    zb = jnp.maximum(zb, 0.0)
    out_ref[...] = (jnp.dot(zb, w2b_ref[...], preferred_element_type=jnp.float32)
                    + b2b_ref[...])


def _final(s, cnt, x, w1b, b1b, w2a, b2a, g2, be2, w2b, b2b):
    N, D = x.shape
    H = w1b.shape[1]
    return pl.pallas_call(
        _final_body,
        out_shape=jax.ShapeDtypeStruct((N, H), jnp.float32),
    )(s, cnt, x, w1b, b1b, w2a, b2a, g2, be2, w2b, b2b)


def kernel(x, edge_index, edge_attr, u, batch,
           W1a, b1a, g1, be1, W1b, b1b,
           W2a, b2a, g2, be2, W2b, b2b):
    N, D = x.shape
    E = edge_index.shape[1]
    H = edge_attr.shape[1]
    row = edge_index[0]
    col = edge_index[1]

    b1a2 = b1a.reshape(1, H)
    g12 = g1.reshape(1, H)
    be12 = be1.reshape(1, H)

    t = _t_matmul(edge_attr, W1a[D:], b1a2, blk=2000)

    ones_e = jnp.ones((E,), jnp.float32)
    cnt_col = jax.ops.segment_sum(ones_e, col, num_segments=N)[:, None]

    p = _pmat(x, W1a[0:D])
    hs = _make_hstats(E, H)(t, row, p)

    einv = 1.0 / E
    p2, scale = _stats(p, g12, be12, hs, einv)

    y = _make_main(E, H)(t, row, p2, scale.reshape(H))
    s = jax.ops.segment_sum(y, col, num_segments=N)

    out = _final(s, cnt_col, x,
                 W1b, b1b.reshape(1, H), W2a, b2a.reshape(1, H),
                 g2.reshape(1, H), be2.reshape(1, H), W2b, b2b.reshape(1, H))
    return out
